# Initial kernel scaffold; baseline (speedup 1.0000x reference)
import jax
import jax.numpy as jnp
from jax.experimental import pallas as pl


def kernel(x):
    raise NotImplementedError("write your pallas kernel here")


# speedup vs baseline: 3.2210x; 3.2210x over previous
import jax
import jax.numpy as jnp
from jax.experimental import pallas as pl

_ROWS_PER_BLOCK = 8
_TOPK = 64


def _topk_mask_kernel(x_ref, o_ref):
    x = x_ref[...]
    r, n = x.shape
    bits = jax.lax.bitcast_convert_type(x, jnp.int32)
    skey = jnp.where(bits < 0, bits ^ jnp.int32(0x7FFFFFFF), bits)
    min32 = jnp.int32(-(2 ** 31))

    def value_bit(i, lu):
        bit = 31 - i
        cand_u = lu | (jnp.int32(1) << bit)
        cand_s = cand_u ^ min32
        cnt = jnp.sum((skey >= cand_s).astype(jnp.int32), axis=1,
                      keepdims=True)
        return jnp.where(cnt >= _TOPK, cand_u, lu)

    lu = jax.lax.fori_loop(0, 32, value_bit, jnp.zeros((r, 1), jnp.int32))
    thresh = lu ^ min32

    gt = skey > thresh
    eq = skey == thresh
    c_gt = jnp.sum(gt.astype(jnp.int32), axis=1, keepdims=True)
    m = _TOPK - c_gt
    idx = jax.lax.broadcasted_iota(jnp.int32, (r, n), 1)

    def index_bit(i, t):
        bit = 14 - i
        cand = t | (jnp.int32(1) << bit)
        cnt = jnp.sum((eq & (idx < cand)).astype(jnp.int32), axis=1,
                      keepdims=True)
        return jnp.where(cnt < m, cand, t)

    t_idx = jax.lax.fori_loop(0, 15, index_bit,
                              jnp.zeros((r, 1), jnp.int32))
    mask = gt | (eq & (idx <= t_idx))
    o_ref[...] = jnp.where(mask, jnp.maximum(x, 0.0), 0.0)


def kernel(x):
    m, n = x.shape
    return pl.pallas_call(
        _topk_mask_kernel,
        grid=(m // _ROWS_PER_BLOCK,),
        in_specs=[pl.BlockSpec((_ROWS_PER_BLOCK, n), lambda i: (i, 0))],
        out_specs=pl.BlockSpec((_ROWS_PER_BLOCK, n), lambda i: (i, 0)),
        out_shape=jax.ShapeDtypeStruct((m, n), x.dtype),
    )(x)


# speedup vs baseline: 6.3739x; 1.9788x over previous
import jax
import jax.numpy as jnp
from jax.experimental import pallas as pl

_ROWS_PER_BLOCK = 8
_TOPK = 64


def _count(pred):
    acc = pred.astype(jnp.int16)
    w = acc.shape[1] // 2
    while w >= 256:
        acc = acc[:, :w] + acc[:, w:]
        w //= 2
    return jnp.sum(acc.astype(jnp.int32), axis=1, keepdims=True)


def _topk_mask_kernel(x_ref, o_ref):
    x = x_ref[...]
    r, n = x.shape
    bits = jax.lax.bitcast_convert_type(x, jnp.int32)
    skey = jnp.where(bits < 0, bits ^ jnp.int32(0x7FFFFFFF), bits)
    hi = jnp.right_shift(skey, 16).astype(jnp.int16)
    lo = ((skey & jnp.int32(0xFFFF)) - 32768).astype(jnp.int16)

    bias = jnp.int32(32768)

    def hi_bit(i, lu):
        bit = 15 - i
        cand_u = lu | (jnp.int32(1) << bit)
        cand_s = (cand_u - bias).astype(jnp.int16)
        cnt = _count(hi >= cand_s)
        return jnp.where(cnt >= _TOPK, cand_u, lu)

    lh = jax.lax.fori_loop(0, 16, hi_bit, jnp.zeros((r, 1), jnp.int32))
    th = (lh - bias).astype(jnp.int16)
    eq_hi = hi == th
    m2 = _TOPK - _count(hi > th)
    lo_m = jnp.where(eq_hi, lo, jnp.int16(-32768))

    def lo_bit(i, lu):
        bit = 15 - i
        cand_u = lu | (jnp.int32(1) << bit)
        cand_s = (cand_u - bias).astype(jnp.int16)
        cnt = _count(lo_m >= cand_s)
        return jnp.where(cnt >= m2, cand_u, lu)

    ll = jax.lax.fori_loop(0, 16, lo_bit, jnp.zeros((r, 1), jnp.int32))
    tl = (ll - bias).astype(jnp.int16)

    eq = eq_hi & (lo == tl)
    gt = (hi > th) | (lo_m > tl)
    m = _TOPK - _count(gt)
    c_eq = _count(eq)

    h = n // 2
    idx = jax.lax.broadcasted_iota(jnp.int16, (r, h), 1)
    idx = jnp.concatenate([idx, idx + jnp.int16(h)], axis=1)
    idx_m = jnp.where(eq, idx, jnp.int16(32767))

    def tie_search(_):
        def index_bit(i, t):
            bit = 14 - i
            cand = (t | (jnp.int32(1) << bit)).astype(jnp.int16)
            cnt = _count(idx_m < cand)
            return jnp.where(cnt < m, cand.astype(jnp.int32), t)

        return jax.lax.fori_loop(0, 15, index_bit,
                                 jnp.zeros((r, 1), jnp.int32))

    t_idx = jax.lax.cond(jnp.all(m == c_eq),
                         lambda _: jnp.full((r, 1), 32767, jnp.int32),
                         tie_search, 0)
    t16 = t_idx.astype(jnp.int16)

    mask = gt | (eq & (idx_m <= t16))
    o_ref[...] = jnp.where(mask, jnp.maximum(x, 0.0), 0.0)


def kernel(x):
    m, n = x.shape
    return pl.pallas_call(
        _topk_mask_kernel,
        grid=(m // _ROWS_PER_BLOCK,),
        in_specs=[pl.BlockSpec((_ROWS_PER_BLOCK, n), lambda i: (i, 0))],
        out_specs=pl.BlockSpec((_ROWS_PER_BLOCK, n), lambda i: (i, 0)),
        out_shape=jax.ShapeDtypeStruct((m, n), x.dtype),
    )(x)
